# baseline (speedup 1.0000x reference)
import functools

import jax
import jax.numpy as jnp
from jax import lax
from jax.experimental import pallas as pl
from jax.experimental.pallas import tpu as pltpu
from jax.experimental.pallas import tpu_sc as plsc

_NC, _NS, _L = 2, 16, 16
_NW = _NC * _NS

_T, _K, _D = 16384, 9, 32
_R = _K - 1
_TW = _T // _NW
_CH = 128
_NCHUNK = _TW // _CH
_ROWS = _CH * _K


def _body(ss_hbm, z_hbm, outq_hbm, outr_hbm, block_v, g9_v, q_v, o_v, sem):
    wid = lax.axis_index("s") * _NC + lax.axis_index("c")
    t0 = wid * _TW
    pltpu.sync_copy(ss_hbm.at[pl.ds(t0 * _K, _TW * _K)], block_v)

    lane = jnp.arange(16, dtype=jnp.int32)
    row_off = []
    col_off = []
    for v in range(16):
        j = v * 16 + lane
        row_off.append(1 + (j & 7))
        col_off.append(j >> 3)

    for c in range(_NCHUNK):
        base = c * _ROWS
        cps = []
        for i in range(_K):
            idx = block_v.at[pl.ds(base + i * 128, 128)]
            cps.append(
                pltpu.async_copy(
                    z_hbm.at[idx], g9_v.at[pl.ds(i * 128, 128)], sem
                )
            )
        for cp in cps:
            cp.wait()

        def trial_body(tl, carry):
            g_base = tl * _K
            for v in range(_D // _L):
                q_v[pl.ds(tl * _D + v * _L, _L)] = g9_v[
                    g_base, pl.ds(v * _L, _L)
                ]
            for v in range(16):
                vals = plsc.load_gather(
                    g9_v, [g_base + row_off[v], col_off[v]]
                )
                o_v[pl.ds(tl * (_D * _R) + v * _L, _L)] = vals
            return carry

        lax.fori_loop(0, _CH, trial_body, 0)

        tc0 = t0 + c * _CH
        pltpu.sync_copy(q_v, outq_hbm.at[pl.ds(tc0 * _D, _CH * _D)])
        pltpu.sync_copy(
            o_v, outr_hbm.at[pl.ds(tc0 * _D * _R, _CH * _D * _R)]
        )


@jax.jit
def _run(ss_flat, z):
    kfn = pl.kernel(
        _body,
        out_type=(
            jax.ShapeDtypeStruct((_T * _D,), jnp.float32),
            jax.ShapeDtypeStruct((_T * _D * _R,), jnp.float32),
        ),
        mesh=plsc.VectorSubcoreMesh(
            core_axis_name="c", subcore_axis_name="s",
            num_cores=_NC, num_subcores=_NS,
        ),
        scratch_types=[
            pltpu.VMEM((_TW * _K,), jnp.int32),
            pltpu.VMEM((_ROWS, _D), jnp.float32),
            pltpu.VMEM((_CH * _D,), jnp.float32),
            pltpu.VMEM((_CH * _D * _R,), jnp.float32),
            pltpu.SemaphoreType.DMA,
        ],
        compiler_params=pltpu.CompilerParams(
            needs_layout_passes=False, use_tc_tiling_on_sc=False
        ),
    )
    return kfn(ss_flat, z)


def kernel(stimulus_set, max_n_reference, z):
    del max_n_reference
    q_flat, r_flat = _run(stimulus_set.reshape(-1), z)
    return (
        q_flat.reshape(_T, _D, 1),
        r_flat.reshape(_T, _D, _R),
    )
